# baseline (speedup 1.0000x reference)
import functools

import jax
import jax.numpy as jnp
from jax import lax
from jax.experimental import pallas as pl
from jax.experimental.pallas import tpu as pltpu
from jax.experimental.pallas import tpu_sc as plsc

B = 256
D = 2048
A = 8
O = 16
R = 2 ** A
L = 16
NC = 2
NS = 16
NW = NC * NS
DW = D // NW
XC = 16
PC = 8
NCH = DW // XC
GROUPS = B // L

_NET = (
    (0, 1), (2, 3), (4, 5), (6, 7),
    (0, 2), (1, 3), (4, 6), (5, 7),
    (1, 2), (5, 6),
    (0, 4), (1, 5), (2, 6), (3, 7),
    (2, 4), (3, 5),
    (1, 2), (3, 4), (5, 6),
)

_mesh = plsc.VectorSubcoreMesh(core_axis_name="c", subcore_axis_name="s")


@functools.partial(
    pl.kernel,
    out_type=jax.ShapeDtypeStruct((B, D * O), jnp.float32),
    mesh=_mesh,
    scratch_types=[
        pltpu.VMEM((XC * A, B), jnp.float32),
        pltpu.VMEM((PC, R * O), jnp.float32),
        pltpu.VMEM((B, PC * O), jnp.float32),
    ],
    compiler_params=pltpu.CompilerParams(needs_layout_passes=False),
)
def _hoa(x_hbm, p_hbm, out_hbm, x_v, p_v, o_v):
    wid = lax.axis_index("s") * NC + lax.axis_index("c")
    d0 = wid * DW

    def chunk_body(ci, carry):
        dc = d0 + ci * XC
        pltpu.sync_copy(x_hbm.at[pl.ds(dc * A, XC * A)], x_v)

        def half_body(h, carry):
            dp = dc + h * PC
            pltpu.sync_copy(p_hbm.at[pl.ds(dp, PC)], p_v)

            def d_body(ds_, carry):
                c0 = (h * PC + ds_) * A
                ocol0 = ds_ * O
                row255 = p_v[ds_, pl.ds(255 * O, O)]

                def g_body(g, carry):
                    gb = g * L
                    vs = [x_v[c0 + k, pl.ds(gb, L)] for k in range(A)]
                    ms = [jnp.full((L,), 1 << k, jnp.int32) for k in range(A)]
                    for (i, j) in _NET:
                        p = vs[i] <= vs[j]
                        lo = jnp.minimum(vs[i], vs[j])
                        hi = jnp.maximum(vs[i], vs[j])
                        ml = jnp.where(p, ms[i], ms[j])
                        mh = jnp.where(p, ms[j], ms[i])
                        vs[i], vs[j], ms[i], ms[j] = lo, hi, ml, mh
                    cs = [vs[0]] + [vs[k] - vs[k - 1] for k in range(1, A)]
                    ind = ms[A - 1]
                    rows = [None] * A
                    rows[A - 1] = ind * O
                    for k in range(A - 2, 0, -1):
                        ind = ind + ms[k]
                        rows[k] = ind * O
                    for b in range(L):
                        acc = cs[0][b] * row255
                        for k in range(1, A):
                            prow = p_v[ds_, pl.ds(rows[k][b], O)]
                            acc = acc + cs[k][b] * prow
                        o_v[gb + b, pl.ds(ocol0, O)] = acc
                    return carry

                lax.fori_loop(0, GROUPS, g_body, 0)
                return carry

            lax.fori_loop(0, PC, d_body, 0)
            pltpu.sync_copy(o_v, out_hbm.at[:, pl.ds(dp * O, PC * O)])
            return carry

        lax.fori_loop(0, 2, half_body, 0)
        return carry

    lax.fori_loop(0, NCH, chunk_body, 0)


def kernel(X, params):
    x_t = X.reshape(B, D * A).T
    out = _hoa(x_t, params.reshape(D, R * O))
    return out.reshape(B, D, O)
